# Initial kernel scaffold; baseline (speedup 1.0000x reference)
import jax
import jax.numpy as jnp
from jax.experimental import pallas as pl


def kernel(x, position, value_table):
    raise NotImplementedError("write your pallas kernel here")


# speedup vs baseline: 4.5131x; 4.5131x over previous
import functools

import jax
import jax.numpy as jnp
from jax.experimental import pallas as pl
from jax.experimental.pallas import tpu as pltpu

B = 16
N_POS = 3072
LEVELS = 256
D = 4096
D_TILE = 512


def _fire_tc_kernel(xf_ref, pos_ref, tab_ref, out_ref, oh_ref):
    @pl.when(pl.program_id(0) == 0)
    def _build_onehot():
        for b in range(B):
            idx = (xf_ref[b:b + 1, :] * float(LEVELS - 1)).astype(jnp.int32)
            lv = jax.lax.broadcasted_iota(jnp.int32, (LEVELS, N_POS), 0)
            oh_ref[pl.ds(b * LEVELS, LEVELS), :] = (lv == idx).astype(jnp.bfloat16)

    q = jnp.dot(oh_ref[:, :], pos_ref[:, :], preferred_element_type=jnp.float32)
    tab = tab_ref[:, :]
    for b in range(B):
        acc = jnp.sum(tab * q[b * LEVELS:(b + 1) * LEVELS, :], axis=0)
        out_ref[b, :] = jnp.where(acc > 0.0, 1.0, -1.0)


@jax.jit
def kernel(x, position, value_table):
    xf = x.reshape(B, N_POS)
    pos_bf = position.astype(jnp.bfloat16)
    grid = (D // D_TILE,)
    return pl.pallas_call(
        _fire_tc_kernel,
        grid=grid,
        in_specs=[
            pl.BlockSpec((B, N_POS), lambda i: (0, 0)),
            pl.BlockSpec((N_POS, D_TILE), lambda i: (0, i)),
            pl.BlockSpec((LEVELS, D_TILE), lambda i: (0, i)),
        ],
        out_specs=pl.BlockSpec((B, D_TILE), lambda i: (0, i)),
        out_shape=jax.ShapeDtypeStruct((B, D), jnp.float32),
        scratch_shapes=[pltpu.VMEM((B * LEVELS, N_POS), jnp.bfloat16)],
    )(xf, pos_bf, value_table)
